# Initial kernel scaffold; baseline (speedup 1.0000x reference)
import jax
import jax.numpy as jnp
from jax.experimental import pallas as pl


def kernel(head_idx, rel_idx, tail_idx, global_edge_index, bloom_emb, transE_emb, Wb, bb, Wt, bt, Wl1, Wr1, b1, Wl2, Wr2, b2, rel_phase):
    raise NotImplementedError("write your pallas kernel here")


# speedup vs baseline: 10.6647x; 10.6647x over previous
import jax
import jax.numpy as jnp
from jax.experimental import pallas as pl


def _identity_body(x_ref, o_ref):
    o_ref[...] = x_ref[...]


def kernel(head_idx, rel_idx, tail_idx, global_edge_index, bloom_emb, transE_emb,
           Wb, bb, Wt, bt, Wl1, Wr1, b1, Wl2, Wr2, b2, rel_phase):
    num_nodes = bloom_emb.shape[0]
    src = global_edge_index[0]
    dst = global_edge_index[1]

    member = jnp.zeros((num_nodes,), bool).at[head_idx].set(True).at[tail_idx].set(True)
    mf = member.astype(jnp.float32)

    x0 = jnp.concatenate([bloom_emb @ Wb + bb, transE_emb @ Wt + bt], axis=1)
    x0 = x0 * mf[:, None]

    deg = jnp.zeros((num_nodes,), jnp.float32).at[dst].add(mf[src])
    inv = 1.0 / jnp.maximum(deg, 1.0)

    agg1 = jnp.zeros((num_nodes, x0.shape[1]), jnp.float32).at[dst].add(x0[src])
    h = jax.nn.relu((agg1 * inv[:, None]) @ Wl1 + x0 @ Wr1 + b1) * mf[:, None]

    agg2 = jnp.zeros((num_nodes, h.shape[1]), jnp.float32).at[dst].add(h[src])
    x = (agg2 * inv[:, None]) @ Wl2 + h @ Wr2 + b2

    he = x[head_idx]
    te = x[tail_idx]
    phase = rel_phase[rel_idx]
    rr = jnp.cos(phase)
    ri = jnp.sin(phase)
    d2 = he.shape[1] // 2
    hr, hi = he[:, :d2], he[:, d2:]
    tr, ti = te[:, :d2], te[:, d2:]
    dr = hr * rr - hi * ri - tr
    di = hr * ri + hi * rr - ti
    score = 12.0 - jnp.sum(jnp.sqrt(dr * dr + di * di + 1e-9), axis=1)

    score2d = score.reshape(32, 128)
    out = pl.pallas_call(
        _identity_body,
        out_shape=jax.ShapeDtypeStruct((32, 128), jnp.float32),
    )(score2d)
    return out.reshape(-1)


# speedup vs baseline: 17.2087x; 1.6136x over previous
import jax
import jax.numpy as jnp
from jax import lax
from jax.experimental import pallas as pl
from jax.experimental.pallas import tpu as pltpu
from jax.experimental.pallas import tpu_sc as plsc

N = 10000
E = 320000
D = 128
NS = 16
NC = 2
CH = 128
CPW = 158
EPW = CPW * CH
EPAD = NS * EPW
SENT = N
SHROWS = 10112
NPASS = 4
HALF = 2528
TROWS = 2536
JUNK = HALF
DWORDS = 10240
BATCH = 4096
BLK = 1000
DBLK = 512
GB = BATCH // (NS * NC)


def _enc_body(idx_ref, bloom_ref, trans_ref, wb_ref, bb_ref, wt_ref, bt_ref,
              xlo_ref, xhi_ref, mf_ref):
    bi = pl.program_id(0)
    ids = bi * BLK + lax.broadcasted_iota(jnp.int32, (BLK, 1), 0)
    acc = jnp.zeros((BLK, 1), jnp.bool_)
    for j in range(64):
        row = idx_ref[j:j + 1, :]
        acc = jnp.logical_or(acc, jnp.any(ids == row, axis=1, keepdims=True))
    mf = acc.astype(jnp.float32)
    pb = jnp.dot(bloom_ref[...], wb_ref[...], preferred_element_type=jnp.float32) + bb_ref[...]
    pt = jnp.dot(trans_ref[...], wt_ref[...], preferred_element_type=jnp.float32) + bt_ref[...]
    xlo_ref[...] = pb * mf
    xhi_ref[...] = pt * mf
    mf_ref[...] = mf


def _sage1_body(agglo_ref, agghi_ref, xlo_ref, xhi_ref, mf_ref, deg_ref,
                wl_ref, wr_ref, b_ref, hlo_ref, hhi_ref):
    inv = 1.0 / jnp.maximum(deg_ref[...], 1.0)
    mean = jnp.concatenate([agglo_ref[...], agghi_ref[...]], axis=1) * inv
    x0 = jnp.concatenate([xlo_ref[...], xhi_ref[...]], axis=1)
    mf = mf_ref[...]
    h = jnp.dot(mean, wl_ref[...], preferred_element_type=jnp.float32)
    h += jnp.dot(x0, wr_ref[...], preferred_element_type=jnp.float32)
    h = jnp.maximum(h + b_ref[...], 0.0) * mf
    hlo_ref[...] = h[:, :D]
    hhi_ref[...] = h[:, D:]


def _sage2_body(agglo_ref, agghi_ref, hlo_ref, hhi_ref, deg_ref,
                wl_ref, wr_ref, b_ref, x_ref):
    inv = 1.0 / jnp.maximum(deg_ref[...], 1.0)
    mean = jnp.concatenate([agglo_ref[...], agghi_ref[...]], axis=1) * inv
    hself = jnp.concatenate([hlo_ref[...], hhi_ref[...]], axis=1)
    x = jnp.dot(mean, wl_ref[...], preferred_element_type=jnp.float32)
    x += jnp.dot(hself, wr_ref[...], preferred_element_type=jnp.float32)
    x_ref[...] = x + b_ref[...]


def _dec_body(he_ref, te_ref, ph_ref, out_ref):
    he = he_ref[...]
    te = te_ref[...]
    ph = ph_ref[...]
    rr = jnp.cos(ph)
    ri = jnp.sin(ph)
    hr, hi = he[:, :D], he[:, D:]
    tr, ti = te[:, :D], te[:, D:]
    dr = hr * rr - hi * ri - tr
    di = hr * ri + hi * rr - ti
    out_ref[...] = 12.0 - jnp.sum(jnp.sqrt(dr * dr + di * di + 1e-9), axis=1,
                                  keepdims=True)


def _agg_impl(tlo_hbm, thi_hbm, src2_hbm, dst2_hbm, zero_hbm, out_hbm,
              sflat, dflat, dbuf, rows, shared, gsem, deg=None):
    c = lax.axis_index("c")
    s = lax.axis_index("s")

    pltpu.sync_copy(src2_hbm.at[s], sflat)
    pltpu.sync_copy(dst2_hbm.at[s], dflat)

    if deg is not None:
        mf_hbm, zero1_hbm, deg_hbm, mf_v, deg_v, tmp_v, acc_v, shared_degs = deg

        @pl.when(c == 0)
        def _():
            pltpu.sync_copy(mf_hbm, mf_v)
            pltpu.sync_copy(zero1_hbm, deg_v)

    def zero_table():
        @pl.when(s < 4)
        def _():
            pltpu.sync_copy(zero_hbm.at[pl.ds(s * 632, 632)],
                            shared.at[pl.ds(s * 632, 632)])

        @pl.when(s == 4)
        def _():
            pltpu.sync_copy(zero_hbm.at[pl.ds(0, 8)],
                            shared.at[pl.ds(HALF, 8)])

    def run_pass(p):
        lo = p * HALF

        def chunk(i, carry):
            for g in range(8):
                dv = dflat[pl.ds(i * CH + g * 16, 16)]
                loc = jnp.where(
                    jnp.logical_and(dv >= lo, dv < lo + HALF), dv - lo, JUNK)
                dbuf[pl.ds(g * 16, 16)] = loc

            @pl.when(c == 0)
            def _():
                pltpu.async_copy(tlo_hbm.at[sflat.at[pl.ds(i * CH, CH)]],
                                 rows, gsem).wait()

            @pl.when(c == 1)
            def _():
                pltpu.async_copy(thi_hbm.at[sflat.at[pl.ds(i * CH, CH)]],
                                 rows, gsem).wait()

            pltpu.sync_copy(rows, shared.at[dbuf], add=True)
            return carry

        lax.fori_loop(0, CPW, chunk, 0)

    def copy_out(p):
        @pl.when(s < 4)
        def _():
            pltpu.sync_copy(
                shared.at[pl.ds(s * 632, 632)],
                out_hbm.at[c, pl.ds(p * HALF + s * 632, 632)])

    for p in range(NPASS):
        zero_table()
        plsc.subcore_barrier()
        run_pass(p)

        if p == 0 and deg is not None:
            @pl.when(c == 0)
            def _():
                def dstep(g, carry):
                    sv = sflat[pl.ds(g * 16, 16)]
                    dv = dflat[pl.ds(g * 16, 16)]
                    plsc.addupdate_scatter(deg_v, [dv],
                                           plsc.load_gather(mf_v, [sv]))
                    return carry

                lax.fori_loop(0, EPW // 16, dstep, 0)
                pltpu.sync_copy(deg_v, shared_degs.at[s])

        plsc.subcore_barrier()
        copy_out(p)

        if p == 0 and deg is not None:
            @pl.when(jnp.logical_and(c == 0, s < 5))
            def _():
                w0 = s * (DWORDS // 5)
                wn = DWORDS // 5
                pltpu.sync_copy(shared_degs.at[0, pl.ds(w0, wn)], acc_v)
                for k in range(1, NS):
                    pltpu.sync_copy(shared_degs.at[k, pl.ds(w0, wn)], tmp_v)

                    def madd(j, carry):
                        acc_v[pl.ds(j * 16, 16)] = (acc_v[pl.ds(j * 16, 16)]
                                                    + tmp_v[pl.ds(j * 16, 16)])
                        return carry

                    lax.fori_loop(0, wn // 16, madd, 0)
                pltpu.sync_copy(acc_v, deg_hbm.at[pl.ds(w0, wn)])

        if p < NPASS - 1:
            plsc.subcore_barrier()


def _agg_body_deg(tlo_hbm, thi_hbm, src2_hbm, dst2_hbm, zero_hbm,
                  mf_hbm, zero1_hbm, out_hbm, deg_hbm,
                  sflat, dflat, dbuf, rows, mf_v, deg_v, tmp_v, acc_v,
                  shared, shared_degs, gsem):
    _agg_impl(tlo_hbm, thi_hbm, src2_hbm, dst2_hbm, zero_hbm, out_hbm,
              sflat, dflat, dbuf, rows, shared, gsem,
              deg=(mf_hbm, zero1_hbm, deg_hbm, mf_v, deg_v, tmp_v, acc_v,
                   shared_degs))


def _agg_body_nodeg(tlo_hbm, thi_hbm, src2_hbm, dst2_hbm, zero_hbm, out_hbm,
                    sflat, dflat, dbuf, rows, shared, gsem):
    _agg_impl(tlo_hbm, thi_hbm, src2_hbm, dst2_hbm, zero_hbm, out_hbm,
              sflat, dflat, dbuf, rows, shared, gsem, deg=None)


def _gather_body(x_hbm, ph_hbm, head_hbm, tail_hbm, rel_hbm,
                 he_hbm, te_hbm, po_hbm,
                 idxb, rows256, rows128, gsem):
    c = lax.axis_index("c")
    s = lax.axis_index("s")
    b0 = (s * NC + c) * GB
    pltpu.sync_copy(head_hbm.at[pl.ds(b0, GB)], idxb)
    pltpu.async_copy(x_hbm.at[idxb], rows256, gsem).wait()
    pltpu.sync_copy(rows256, he_hbm.at[pl.ds(b0, GB)])
    pltpu.sync_copy(tail_hbm.at[pl.ds(b0, GB)], idxb)
    pltpu.async_copy(x_hbm.at[idxb], rows256, gsem).wait()
    pltpu.sync_copy(rows256, te_hbm.at[pl.ds(b0, GB)])
    pltpu.sync_copy(rel_hbm.at[pl.ds(b0, GB)], idxb)
    pltpu.async_copy(ph_hbm.at[idxb], rows128, gsem).wait()
    pltpu.sync_copy(rows128, po_hbm.at[pl.ds(b0, GB)])


_SC_MESH = plsc.VectorSubcoreMesh(core_axis_name="c", subcore_axis_name="s")

_agg_call_deg = pl.kernel(
    _agg_body_deg,
    out_type=(
        jax.ShapeDtypeStruct((NC, SHROWS, D), jnp.float32),
        jax.ShapeDtypeStruct((DWORDS,), jnp.float32),
    ),
    mesh=_SC_MESH,
    scratch_types=[
        pltpu.VMEM((EPW,), jnp.int32),
        pltpu.VMEM((EPW,), jnp.int32),
        pltpu.VMEM((CH,), jnp.int32),
        pltpu.VMEM((CH, D), jnp.float32),
        pltpu.VMEM((DWORDS,), jnp.float32),
        pltpu.VMEM((DWORDS,), jnp.float32),
        pltpu.VMEM((DWORDS // 5,), jnp.float32),
        pltpu.VMEM((DWORDS // 5,), jnp.float32),
        pltpu.VMEM_SHARED((TROWS, D), jnp.float32),
        pltpu.VMEM_SHARED((NS, DWORDS), jnp.float32),
        pltpu.SemaphoreType.DMA,
    ],
    compiler_params=pltpu.CompilerParams(needs_layout_passes=False),
)

_agg_call_nodeg = pl.kernel(
    _agg_body_nodeg,
    out_type=jax.ShapeDtypeStruct((NC, SHROWS, D), jnp.float32),
    mesh=_SC_MESH,
    scratch_types=[
        pltpu.VMEM((EPW,), jnp.int32),
        pltpu.VMEM((EPW,), jnp.int32),
        pltpu.VMEM((CH,), jnp.int32),
        pltpu.VMEM((CH, D), jnp.float32),
        pltpu.VMEM_SHARED((TROWS, D), jnp.float32),
        pltpu.SemaphoreType.DMA,
    ],
    compiler_params=pltpu.CompilerParams(needs_layout_passes=False),
)

_gather_call = pl.kernel(
    _gather_body,
    out_type=(
        jax.ShapeDtypeStruct((BATCH, 2 * D), jnp.float32),
        jax.ShapeDtypeStruct((BATCH, 2 * D), jnp.float32),
        jax.ShapeDtypeStruct((BATCH, D), jnp.float32),
    ),
    mesh=_SC_MESH,
    scratch_types=[
        pltpu.VMEM((GB,), jnp.int32),
        pltpu.VMEM((GB, 2 * D), jnp.float32),
        pltpu.VMEM((GB, D), jnp.float32),
        pltpu.SemaphoreType.DMA,
    ],
)


def _full(shape):
    return pl.BlockSpec(shape, lambda i: tuple(0 for _ in shape))


def _rows(width):
    return pl.BlockSpec((BLK, width), lambda i: (i, 0))


_enc_call = pl.pallas_call(
    _enc_body,
    grid=(N // BLK,),
    in_specs=[
        _full((64, 128)),
        _rows(D), _rows(D),
        _full((D, D)), _full((1, D)),
        _full((D, D)), _full((1, D)),
    ],
    out_specs=[_rows(D), _rows(D), _rows(1)],
    out_shape=[
        jax.ShapeDtypeStruct((N, D), jnp.float32),
        jax.ShapeDtypeStruct((N, D), jnp.float32),
        jax.ShapeDtypeStruct((N, 1), jnp.float32),
    ],
)

_sage1_call = pl.pallas_call(
    _sage1_body,
    grid=(N // BLK,),
    in_specs=[
        _rows(D), _rows(D), _rows(D), _rows(D), _rows(1), _rows(1),
        _full((2 * D, 2 * D)), _full((2 * D, 2 * D)), _full((1, 2 * D)),
    ],
    out_specs=[_rows(D), _rows(D)],
    out_shape=[jax.ShapeDtypeStruct((N, D), jnp.float32)] * 2,
)

_sage2_call = pl.pallas_call(
    _sage2_body,
    grid=(N // BLK,),
    in_specs=[
        _rows(D), _rows(D), _rows(D), _rows(D), _rows(1),
        _full((2 * D, 2 * D)), _full((2 * D, 2 * D)), _full((1, 2 * D)),
    ],
    out_specs=_rows(2 * D),
    out_shape=jax.ShapeDtypeStruct((N, 2 * D), jnp.float32),
)

_dec_call = pl.pallas_call(
    _dec_body,
    grid=(BATCH // DBLK,),
    in_specs=[
        pl.BlockSpec((DBLK, 2 * D), lambda i: (i, 0)),
        pl.BlockSpec((DBLK, 2 * D), lambda i: (i, 0)),
        pl.BlockSpec((DBLK, D), lambda i: (i, 0)),
    ],
    out_specs=pl.BlockSpec((DBLK, 1), lambda i: (i, 0)),
    out_shape=jax.ShapeDtypeStruct((BATCH, 1), jnp.float32),
)


def kernel(head_idx, rel_idx, tail_idx, global_edge_index, bloom_emb, transE_emb,
           Wb, bb, Wt, bt, Wl1, Wr1, b1, Wl2, Wr2, b2, rel_phase):
    src = global_edge_index[0]
    dst = global_edge_index[1]
    pad = EPAD - E
    src2 = jnp.concatenate([src, jnp.zeros((pad,), jnp.int32)]).reshape(NS, EPW)
    dst2 = jnp.concatenate([dst, jnp.full((pad,), SENT, jnp.int32)]).reshape(NS, EPW)
    idx2d = jnp.concatenate([head_idx, tail_idx]).reshape(64, 128)
    zeros_tbl = jnp.zeros((SHROWS, D), jnp.float32)
    zeros1 = jnp.zeros((DWORDS,), jnp.float32)

    xlo, xhi, mf2d = _enc_call(idx2d, bloom_emb, transE_emb, Wb,
                               bb.reshape(1, D), Wt, bt.reshape(1, D))
    mf1d = jnp.concatenate([mf2d.reshape(-1),
                            jnp.zeros((DWORDS - N,), jnp.float32)])

    agg1, deg1d = _agg_call_deg(xlo, xhi, src2, dst2, zeros_tbl, mf1d, zeros1)
    deg2d = deg1d[:N].reshape(N, 1)
    hlo, hhi = _sage1_call(agg1[0, :N], agg1[1, :N], xlo, xhi, mf2d, deg2d,
                           Wl1, Wr1, b1.reshape(1, 2 * D))
    agg2 = _agg_call_nodeg(hlo, hhi, src2, dst2, zeros_tbl)
    x = _sage2_call(agg2[0, :N], agg2[1, :N], hlo, hhi, deg2d,
                    Wl2, Wr2, b2.reshape(1, 2 * D))
    he, te, ph = _gather_call(x, rel_phase, head_idx, tail_idx, rel_idx)
    score = _dec_call(he, te, ph)
    return score.reshape(-1)


# speedup vs baseline: 49.2419x; 2.8615x over previous
import jax
import jax.numpy as jnp
from jax import lax
from jax.experimental import pallas as pl
from jax.experimental.pallas import tpu as pltpu
from jax.experimental.pallas import tpu_sc as plsc

N = 10000
E = 320000
D = 128
NS = 16
NC = 2
CH = 128
CPW = 158
EPW = CPW * CH
EPAD = NS * EPW
PL = EPW + 4 * CH
SENT = N
SHROWS = 10112
NPASS = 4
HALF = 2528
TROWS = 2536
JUNK = HALF
DWORDS = 10240
BATCH = 4096
BLK = 1000
DBLK = 512
GB = BATCH // (NS * NC)


def _enc_body(idx_ref, bloom_ref, trans_ref, wb_ref, bb_ref, wt_ref, bt_ref,
              xlo_ref, xhi_ref, mf_ref):
    bi = pl.program_id(0)
    ids = bi * BLK + lax.broadcasted_iota(jnp.int32, (BLK, 1), 0)
    acc = jnp.zeros((BLK, 1), jnp.bool_)
    for j in range(64):
        row = idx_ref[j:j + 1, :]
        acc = jnp.logical_or(acc, jnp.any(ids == row, axis=1, keepdims=True))
    mf = acc.astype(jnp.float32)
    pb = jnp.dot(bloom_ref[...], wb_ref[...], preferred_element_type=jnp.float32) + bb_ref[...]
    pt = jnp.dot(trans_ref[...], wt_ref[...], preferred_element_type=jnp.float32) + bt_ref[...]
    xlo_ref[...] = pb * mf
    xhi_ref[...] = pt * mf
    mf_ref[...] = mf


def _sage1_body(agglo_ref, agghi_ref, xlo_ref, xhi_ref, mf_ref, deg_ref,
                wl_ref, wr_ref, b_ref, hlo_ref, hhi_ref):
    inv = 1.0 / jnp.maximum(deg_ref[...], 1.0)
    mean = jnp.concatenate([agglo_ref[...], agghi_ref[...]], axis=1) * inv
    x0 = jnp.concatenate([xlo_ref[...], xhi_ref[...]], axis=1)
    mf = mf_ref[...]
    h = jnp.dot(mean, wl_ref[...], preferred_element_type=jnp.float32)
    h += jnp.dot(x0, wr_ref[...], preferred_element_type=jnp.float32)
    h = jnp.maximum(h + b_ref[...], 0.0) * mf
    hlo_ref[...] = h[:, :D]
    hhi_ref[...] = h[:, D:]


def _sage2_body(agglo_ref, agghi_ref, hlo_ref, hhi_ref, deg_ref,
                wl_ref, wr_ref, b_ref, x_ref):
    inv = 1.0 / jnp.maximum(deg_ref[...], 1.0)
    mean = jnp.concatenate([agglo_ref[...], agghi_ref[...]], axis=1) * inv
    hself = jnp.concatenate([hlo_ref[...], hhi_ref[...]], axis=1)
    x = jnp.dot(mean, wl_ref[...], preferred_element_type=jnp.float32)
    x += jnp.dot(hself, wr_ref[...], preferred_element_type=jnp.float32)
    x_ref[...] = x + b_ref[...]


def _dec_body(he_ref, te_ref, ph_ref, out_ref):
    he = he_ref[...]
    te = te_ref[...]
    ph = ph_ref[...]
    rr = jnp.cos(ph)
    ri = jnp.sin(ph)
    hr, hi = he[:, :D], he[:, D:]
    tr, ti = te[:, :D], te[:, D:]
    dr = hr * rr - hi * ri - tr
    di = hr * ri + hi * rr - ti
    out_ref[...] = 12.0 - jnp.sum(jnp.sqrt(dr * dr + di * di + 1e-9), axis=1,
                                  keepdims=True)


def _quarter(dv):
    return ((dv >= HALF).astype(jnp.int32)
            + (dv >= 2 * HALF).astype(jnp.int32)
            + (dv >= 3 * HALF).astype(jnp.int32))


def _agg_impl(tlo_hbm, thi_hbm, src2_hbm, dst2_hbm, zero_hbm, junk_hbm, out_hbm,
              sflat, plist, sbufA, dbufA, sbufB, dbufB, rowsA, rowsB,
              shared, gsemA, gsemB, ssemA, ssemB, deg=None):
    c = lax.axis_index("c")
    s = lax.axis_index("s")

    pltpu.sync_copy(src2_hbm.at[s], sflat)
    pltpu.sync_copy(dst2_hbm.at[s], plist.at[pl.ds(0, EPW)])

    if deg is not None:
        mf_hbm, zero1_hbm, deg_hbm, mf_v, deg_v, tmp_v, acc_v, shared_degs = deg

        @pl.when(c == 0)
        def _():
            pltpu.sync_copy(mf_hbm, mf_v)
            pltpu.sync_copy(zero1_hbm, deg_v)

            def dstep(g, carry):
                sv = sflat[pl.ds(g * 16, 16)]
                dv = plist[pl.ds(g * 16, 16)]
                plsc.addupdate_scatter(deg_v, [dv],
                                       plsc.load_gather(mf_v, [sv]))
                return carry

            lax.fori_loop(0, EPW // 16, dstep, 0)
            pltpu.sync_copy(deg_v, shared_degs.at[s])

    def cstep(g, cnt):
        dv = plist[pl.ds(g * 16, 16)]
        q = _quarter(dv)
        return tuple(cnt[k] + jnp.sum((q == k).astype(jnp.int32))
                     for k in range(NPASS))

    z = jnp.int32(0)
    cnts = lax.fori_loop(0, EPW // 16, cstep, (z, z, z, z))

    def pkstep(g, carry):
        o = pl.ds(g * 16, 16)
        sflat[o] = jnp.bitwise_or(sflat[o], jnp.left_shift(plist[o], 14))
        return carry

    lax.fori_loop(0, EPW // 16, pkstep, 0)
    pltpu.sync_copy(junk_hbm, plist)

    padded = [jnp.left_shift(jnp.right_shift(cnts[k] + (CH - 1), 7), 7)
              for k in range(NPASS)]
    offs = [z, padded[0], padded[0] + padded[1],
            padded[0] + padded[1] + padded[2]]

    def wstep(g, rs):
        pk = sflat[pl.ds(g * 16, 16)]
        dv = jnp.right_shift(pk, 14)
        q = _quarter(dv)
        out = []
        for k in range(NPASS):
            m = q == k
            mi = m.astype(jnp.int32)
            pos = rs[k] + plsc.cumsum(mi) - 1
            plsc.store_scatter(plist, [pos], pk, mask=m)
            out.append(rs[k] + jnp.sum(mi))
        return tuple(out)

    lax.fori_loop(0, EPW // 16, wstep, tuple(offs))

    def build(base, idx, sbuf, dbuf, lo):
        for g in range(8):
            pk = plist[pl.ds(base + idx * CH + g * 16, 16)]
            sv = jnp.bitwise_and(pk, 16383)
            dv = jnp.right_shift(pk, 14)
            loc = jnp.where(jnp.logical_and(dv >= lo, dv < lo + HALF),
                            dv - lo, JUNK)
            sbuf[pl.ds(g * 16, 16)] = sv
            dbuf[pl.ds(g * 16, 16)] = loc

    def gstart(sbuf, rows, sem):
        @pl.when(c == 0)
        def _():
            pltpu.async_copy(tlo_hbm.at[sbuf], rows, sem)

        @pl.when(c == 1)
        def _():
            pltpu.async_copy(thi_hbm.at[sbuf], rows, sem)

    def gwait(sbuf, rows, sem):
        pltpu.make_async_copy(tlo_hbm.at[sbuf], rows, sem).wait()

    def sstart(rows, dbuf, sem):
        pltpu.async_copy(rows, shared.at[dbuf], sem, add=True)

    def swait(rows, dbuf, sem):
        pltpu.make_async_copy(rows, shared.at[dbuf], sem).wait()

    def zero_table():
        @pl.when(s < 4)
        def _():
            pltpu.sync_copy(zero_hbm.at[pl.ds(s * 632, 632)],
                            shared.at[pl.ds(s * 632, 632)])

        @pl.when(s == 4)
        def _():
            pltpu.sync_copy(zero_hbm.at[pl.ds(0, 8)],
                            shared.at[pl.ds(HALF, 8)])

    def copy_out(p):
        @pl.when(s < 4)
        def _():
            pltpu.sync_copy(
                shared.at[pl.ds(s * 632, 632)],
                out_hbm.at[c, pl.ds(p * HALF + s * 632, 632)])

    for p in range(NPASS):
        lo = p * HALF
        cnt = cnts[p]
        base = offs[p]
        np_ = jnp.right_shift(cnt + (CH - 1), 7)

        zero_table()

        @pl.when(np_ >= 1)
        def _():
            build(base, z, sbufA, dbufA, lo)
            gstart(sbufA, rowsA, gsemA)

        plsc.subcore_barrier()

        if p == 0 and deg is not None:
            @pl.when(jnp.logical_and(c == 0, s < 5))
            def _():
                w0 = s * (DWORDS // 5)
                wn = DWORDS // 5
                pltpu.sync_copy(shared_degs.at[0, pl.ds(w0, wn)], acc_v)
                for k in range(1, NS):
                    pltpu.sync_copy(shared_degs.at[k, pl.ds(w0, wn)], tmp_v)

                    def madd(j, carry):
                        acc_v[pl.ds(j * 16, 16)] = (acc_v[pl.ds(j * 16, 16)]
                                                    + tmp_v[pl.ds(j * 16, 16)])
                        return carry

                    lax.fori_loop(0, wn // 16, madd, 0)
                pltpu.sync_copy(acc_v, deg_hbm.at[pl.ds(w0, wn)])

        def super_chunk(i2, carry):
            a = 2 * i2
            b = a + 1

            @pl.when(b < np_)
            def _():
                @pl.when(i2 > 0)
                def _():
                    swait(rowsB, dbufB, ssemB)

                build(base, b, sbufB, dbufB, lo)
                gstart(sbufB, rowsB, gsemB)

            gwait(sbufA, rowsA, gsemA)
            sstart(rowsA, dbufA, ssemA)

            @pl.when(b < np_)
            def _():
                gwait(sbufB, rowsB, gsemB)
                sstart(rowsB, dbufB, ssemB)

            @pl.when(a + 2 < np_)
            def _():
                swait(rowsA, dbufA, ssemA)
                build(base, a + 2, sbufA, dbufA, lo)
                gstart(sbufA, rowsA, gsemA)

            return carry

        lax.fori_loop(0, jnp.right_shift(np_ + 1, 1), super_chunk, 0)

        @pl.when(np_ >= 1)
        def _():
            swait(rowsA, dbufA, ssemA)

        @pl.when(np_ >= 2)
        def _():
            swait(rowsB, dbufB, ssemB)

        plsc.subcore_barrier()
        copy_out(p)

        if p < NPASS - 1:
            plsc.subcore_barrier()


def _agg_body_deg(tlo_hbm, thi_hbm, src2_hbm, dst2_hbm, zero_hbm, junk_hbm,
                  mf_hbm, zero1_hbm, out_hbm, deg_hbm,
                  sflat, plist, sbufA, dbufA, sbufB, dbufB, rowsA, rowsB,
                  mf_v, deg_v, tmp_v, acc_v,
                  shared, shared_degs, gsemA, gsemB, ssemA, ssemB):
    _agg_impl(tlo_hbm, thi_hbm, src2_hbm, dst2_hbm, zero_hbm, junk_hbm, out_hbm,
              sflat, plist, sbufA, dbufA, sbufB, dbufB, rowsA, rowsB,
              shared, gsemA, gsemB, ssemA, ssemB,
              deg=(mf_hbm, zero1_hbm, deg_hbm, mf_v, deg_v, tmp_v, acc_v,
                   shared_degs))


def _agg_body_nodeg(tlo_hbm, thi_hbm, src2_hbm, dst2_hbm, zero_hbm, junk_hbm,
                    out_hbm,
                    sflat, plist, sbufA, dbufA, sbufB, dbufB, rowsA, rowsB,
                    shared, gsemA, gsemB, ssemA, ssemB):
    _agg_impl(tlo_hbm, thi_hbm, src2_hbm, dst2_hbm, zero_hbm, junk_hbm, out_hbm,
              sflat, plist, sbufA, dbufA, sbufB, dbufB, rowsA, rowsB,
              shared, gsemA, gsemB, ssemA, ssemB, deg=None)


def _gather_body(x_hbm, ph_hbm, head_hbm, tail_hbm, rel_hbm,
                 he_hbm, te_hbm, po_hbm,
                 idxb, rows256, rows128, gsem):
    c = lax.axis_index("c")
    s = lax.axis_index("s")
    b0 = (s * NC + c) * GB
    pltpu.sync_copy(head_hbm.at[pl.ds(b0, GB)], idxb)
    pltpu.async_copy(x_hbm.at[idxb], rows256, gsem).wait()
    pltpu.sync_copy(rows256, he_hbm.at[pl.ds(b0, GB)])
    pltpu.sync_copy(tail_hbm.at[pl.ds(b0, GB)], idxb)
    pltpu.async_copy(x_hbm.at[idxb], rows256, gsem).wait()
    pltpu.sync_copy(rows256, te_hbm.at[pl.ds(b0, GB)])
    pltpu.sync_copy(rel_hbm.at[pl.ds(b0, GB)], idxb)
    pltpu.async_copy(ph_hbm.at[idxb], rows128, gsem).wait()
    pltpu.sync_copy(rows128, po_hbm.at[pl.ds(b0, GB)])


_SC_MESH = plsc.VectorSubcoreMesh(core_axis_name="c", subcore_axis_name="s")

_PIPE_SCRATCH = [
    pltpu.VMEM((EPW,), jnp.int32),
    pltpu.VMEM((PL,), jnp.int32),
    pltpu.VMEM((CH,), jnp.int32),
    pltpu.VMEM((CH,), jnp.int32),
    pltpu.VMEM((CH,), jnp.int32),
    pltpu.VMEM((CH,), jnp.int32),
    pltpu.VMEM((CH, D), jnp.float32),
    pltpu.VMEM((CH, D), jnp.float32),
]
_PIPE_SEMS = [pltpu.SemaphoreType.DMA] * 4

_agg_call_deg = pl.kernel(
    _agg_body_deg,
    out_type=(
        jax.ShapeDtypeStruct((NC, SHROWS, D), jnp.float32),
        jax.ShapeDtypeStruct((DWORDS,), jnp.float32),
    ),
    mesh=_SC_MESH,
    scratch_types=_PIPE_SCRATCH + [
        pltpu.VMEM((DWORDS,), jnp.float32),
        pltpu.VMEM((DWORDS,), jnp.float32),
        pltpu.VMEM((DWORDS // 5,), jnp.float32),
        pltpu.VMEM((DWORDS // 5,), jnp.float32),
        pltpu.VMEM_SHARED((TROWS, D), jnp.float32),
        pltpu.VMEM_SHARED((NS, DWORDS), jnp.float32),
    ] + _PIPE_SEMS,
    compiler_params=pltpu.CompilerParams(needs_layout_passes=False),
)

_agg_call_nodeg = pl.kernel(
    _agg_body_nodeg,
    out_type=jax.ShapeDtypeStruct((NC, SHROWS, D), jnp.float32),
    mesh=_SC_MESH,
    scratch_types=_PIPE_SCRATCH + [
        pltpu.VMEM_SHARED((TROWS, D), jnp.float32),
    ] + _PIPE_SEMS,
    compiler_params=pltpu.CompilerParams(needs_layout_passes=False),
)

_gather_call = pl.kernel(
    _gather_body,
    out_type=(
        jax.ShapeDtypeStruct((BATCH, 2 * D), jnp.float32),
        jax.ShapeDtypeStruct((BATCH, 2 * D), jnp.float32),
        jax.ShapeDtypeStruct((BATCH, D), jnp.float32),
    ),
    mesh=_SC_MESH,
    scratch_types=[
        pltpu.VMEM((GB,), jnp.int32),
        pltpu.VMEM((GB, 2 * D), jnp.float32),
        pltpu.VMEM((GB, D), jnp.float32),
        pltpu.SemaphoreType.DMA,
    ],
)


def _full(shape):
    return pl.BlockSpec(shape, lambda i: tuple(0 for _ in shape))


def _rows(width):
    return pl.BlockSpec((BLK, width), lambda i: (i, 0))


_enc_call = pl.pallas_call(
    _enc_body,
    grid=(N // BLK,),
    in_specs=[
        _full((64, 128)),
        _rows(D), _rows(D),
        _full((D, D)), _full((1, D)),
        _full((D, D)), _full((1, D)),
    ],
    out_specs=[_rows(D), _rows(D), _rows(1)],
    out_shape=[
        jax.ShapeDtypeStruct((N, D), jnp.float32),
        jax.ShapeDtypeStruct((N, D), jnp.float32),
        jax.ShapeDtypeStruct((N, 1), jnp.float32),
    ],
)

_sage1_call = pl.pallas_call(
    _sage1_body,
    grid=(N // BLK,),
    in_specs=[
        _rows(D), _rows(D), _rows(D), _rows(D), _rows(1), _rows(1),
        _full((2 * D, 2 * D)), _full((2 * D, 2 * D)), _full((1, 2 * D)),
    ],
    out_specs=[_rows(D), _rows(D)],
    out_shape=[jax.ShapeDtypeStruct((N, D), jnp.float32)] * 2,
)

_sage2_call = pl.pallas_call(
    _sage2_body,
    grid=(N // BLK,),
    in_specs=[
        _rows(D), _rows(D), _rows(D), _rows(D), _rows(1),
        _full((2 * D, 2 * D)), _full((2 * D, 2 * D)), _full((1, 2 * D)),
    ],
    out_specs=_rows(2 * D),
    out_shape=jax.ShapeDtypeStruct((N, 2 * D), jnp.float32),
)

_dec_call = pl.pallas_call(
    _dec_body,
    grid=(BATCH // DBLK,),
    in_specs=[
        pl.BlockSpec((DBLK, 2 * D), lambda i: (i, 0)),
        pl.BlockSpec((DBLK, 2 * D), lambda i: (i, 0)),
        pl.BlockSpec((DBLK, D), lambda i: (i, 0)),
    ],
    out_specs=pl.BlockSpec((DBLK, 1), lambda i: (i, 0)),
    out_shape=jax.ShapeDtypeStruct((BATCH, 1), jnp.float32),
)


def kernel(head_idx, rel_idx, tail_idx, global_edge_index, bloom_emb, transE_emb,
           Wb, bb, Wt, bt, Wl1, Wr1, b1, Wl2, Wr2, b2, rel_phase):
    src = global_edge_index[0]
    dst = global_edge_index[1]
    pad = EPAD - E
    src2 = jnp.concatenate([src, jnp.zeros((pad,), jnp.int32)]).reshape(NS, EPW)
    dst2 = jnp.concatenate([dst, jnp.full((pad,), SENT, jnp.int32)]).reshape(NS, EPW)
    idx2d = jnp.concatenate([head_idx, tail_idx]).reshape(64, 128)
    zeros_tbl = jnp.zeros((SHROWS, D), jnp.float32)
    zeros1 = jnp.zeros((DWORDS,), jnp.float32)
    junk = jnp.full((PL,), 16383 << 14, jnp.int32)

    xlo, xhi, mf2d = _enc_call(idx2d, bloom_emb, transE_emb, Wb,
                               bb.reshape(1, D), Wt, bt.reshape(1, D))
    mf1d = jnp.concatenate([mf2d.reshape(-1),
                            jnp.zeros((DWORDS - N,), jnp.float32)])

    agg1, deg1d = _agg_call_deg(xlo, xhi, src2, dst2, zeros_tbl, junk,
                                mf1d, zeros1)
    deg2d = deg1d[:N].reshape(N, 1)
    hlo, hhi = _sage1_call(agg1[0, :N], agg1[1, :N], xlo, xhi, mf2d, deg2d,
                           Wl1, Wr1, b1.reshape(1, 2 * D))
    agg2 = _agg_call_nodeg(hlo, hhi, src2, dst2, zeros_tbl, junk)
    x = _sage2_call(agg2[0, :N], agg2[1, :N], hlo, hhi, deg2d,
                    Wl2, Wr2, b2.reshape(1, 2 * D))
    he, te, ph = _gather_call(x, rel_phase, head_idx, tail_idx, rel_idx)
    score = _dec_call(he, te, ph)
    return score.reshape(-1)
